# Initial kernel scaffold; baseline (speedup 1.0000x reference)
import jax
import jax.numpy as jnp
from jax.experimental import pallas as pl


def kernel(z, emb):
    raise NotImplementedError("write your pallas kernel here")


# speedup vs baseline: 1.7121x; 1.7121x over previous
import functools

import jax
import jax.numpy as jnp
from jax import lax
from jax.experimental import pallas as pl
from jax.experimental.pallas import tpu as pltpu
from jax.experimental.pallas import tpu_sc as plsc

N_CODES = 1024
DIM = 64
ROWS = 18432
BLK = 1152
NB = ROWS // BLK
BETA_C = 0.25


def _argmin_body(zf_ref, embT_ref, idx_ref, dsum_ref):
    zb = zf_ref[...]
    embT = embT_ref[...]
    m = lax.dot_general(zb, embT,
                        dimension_numbers=(((1,), (0,)), ((), ())),
                        preferred_element_type=jnp.float32)
    z_sq = jnp.sum(zb * zb, axis=1, keepdims=True)
    e_sq = jnp.sum(embT * embT, axis=0, keepdims=True)
    dist = z_sq - 2.0 * m + e_sq
    dmin = jnp.min(dist, axis=1, keepdims=True)
    iota = lax.broadcasted_iota(jnp.int32, dist.shape, 1)
    cand = jnp.where(dist == dmin, iota, jnp.int32(N_CODES))
    idx_ref[0, 0, :] = jnp.min(cand, axis=1)
    dsum_ref[0, 0, :] = jnp.broadcast_to(jnp.sum(dmin), (128,))


def _tc_argmin(z_flat, embT):
    return pl.pallas_call(
        _argmin_body,
        grid=(NB,),
        in_specs=[
            pl.BlockSpec((BLK, DIM), lambda i: (i, 0)),
            pl.BlockSpec((DIM, N_CODES), lambda i: (0, 0)),
        ],
        out_specs=[
            pl.BlockSpec((1, 1, BLK), lambda i: (i, 0, 0)),
            pl.BlockSpec((1, 1, 128), lambda i: (i, 0, 0)),
        ],
        out_shape=[
            jax.ShapeDtypeStruct((NB, 1, BLK), jnp.int32),
            jax.ShapeDtypeStruct((NB, 1, 128), jnp.float32),
        ],
    )(z_flat, embT)


def _sc_gather(emb_pad, idx):
    info = plsc.get_sparse_core_info()
    nc, ns = info.num_cores, info.num_subcores
    nw = nc * ns
    b_per_w = ROWS // nw
    mesh = plsc.VectorSubcoreMesh(core_axis_name="c", subcore_axis_name="s")

    @functools.partial(
        pl.kernel, mesh=mesh,
        out_type=jax.ShapeDtypeStruct((ROWS, 128), jnp.float32),
        scratch_types=[
            pltpu.VMEM((b_per_w,), jnp.int32),
            pltpu.VMEM((b_per_w, 128), jnp.float32),
            pltpu.SemaphoreType.DMA,
        ],
    )
    def gather_k(emb_hbm, idx_hbm, out_hbm, idx_v, rows_v, sem):
        wid = lax.axis_index("s") * nc + lax.axis_index("c")
        base = wid * b_per_w
        pltpu.sync_copy(idx_hbm.at[pl.ds(base, b_per_w)], idx_v)
        pltpu.async_copy(emb_hbm.at[idx_v], rows_v, sem).wait()
        pltpu.sync_copy(rows_v, out_hbm.at[pl.ds(base, b_per_w)])

    return gather_k(emb_pad, idx)


def kernel(z, emb):
    z_flat = jnp.transpose(z, (0, 2, 3, 1)).reshape(ROWS, DIM)
    embT = emb.T
    idx3, dsum3 = _tc_argmin(z_flat, embT)
    idx = idx3.reshape(ROWS)
    emb_pad = jnp.pad(emb, ((0, 0), (0, 128 - DIM)))
    qe = _sc_gather(emb_pad, idx)[:, :DIM]
    quantized_st = qe.reshape(z.shape)
    total = jnp.sum(dsum3[:, 0, 0])
    codebook_loss = (total / jnp.float32(ROWS * DIM)).astype(jnp.float32)
    commitment_loss = (jnp.float32(BETA_C) * codebook_loss).astype(jnp.float32)
    return (quantized_st, commitment_loss, codebook_loss,
            jnp.float32(0.0), jnp.float32(0.0))


# speedup vs baseline: 1.9269x; 1.1255x over previous
import functools

import jax
import jax.numpy as jnp
from jax import lax
from jax.experimental import pallas as pl
from jax.experimental.pallas import tpu as pltpu
from jax.experimental.pallas import tpu_sc as plsc

N_CODES = 1024
DIM = 64
ROWS = 18432
HW = 576
BLK = 1152
NB = ROWS // BLK
BETA_C = 0.25


def _argmin_body(zf_ref, embT_ref, idx_ref, dsum_ref):
    zb = zf_ref[...]
    embT = embT_ref[...]
    m = lax.dot_general(zb, embT,
                        dimension_numbers=(((1,), (0,)), ((), ())),
                        preferred_element_type=jnp.float32)
    z_sq = jnp.sum(zb * zb, axis=1, keepdims=True)
    e_sq = jnp.sum(embT * embT, axis=0, keepdims=True)
    dist = z_sq - 2.0 * m + e_sq
    dmin = jnp.min(dist, axis=1, keepdims=True)
    iota = lax.broadcasted_iota(jnp.int32, dist.shape, 1).astype(jnp.float32)
    cand = jnp.where(dist == dmin, iota, jnp.float32(N_CODES))
    idx_ref[0, 0, :] = jnp.min(cand, axis=1).astype(jnp.int32)
    dsum_ref[0, 0, :] = jnp.broadcast_to(jnp.sum(dmin), (128,))


def _tc_argmin(z_flat, embT):
    return pl.pallas_call(
        _argmin_body,
        grid=(NB,),
        in_specs=[
            pl.BlockSpec((BLK, DIM), lambda i: (i, 0)),
            pl.BlockSpec((DIM, N_CODES), lambda i: (0, 0)),
        ],
        out_specs=[
            pl.BlockSpec((1, 1, BLK), lambda i: (i, 0, 0)),
            pl.BlockSpec((1, 1, 128), lambda i: (i, 0, 0)),
        ],
        out_shape=[
            jax.ShapeDtypeStruct((NB, 1, BLK), jnp.int32),
            jax.ShapeDtypeStruct((NB, 1, 128), jnp.float32),
        ],
    )(z_flat, embT)


def _sc_gather_view(emb_pad, idxp):
    info = plsc.get_sparse_core_info()
    nc, ns = info.num_cores, info.num_subcores
    assert nc * ns == 32
    mesh = plsc.VectorSubcoreMesh(core_axis_name="c", subcore_axis_name="s")

    @functools.partial(
        pl.kernel, mesh=mesh,
        compiler_params=pltpu.CompilerParams(needs_layout_passes=False),
        out_type=jax.ShapeDtypeStruct((32, HW, 128), jnp.float32),
        scratch_types=[
            pltpu.VMEM((HW,), jnp.int32),
            pltpu.VMEM((HW, 128), jnp.float32),
            pltpu.VMEM((64, 128), jnp.float32),
            pltpu.SemaphoreType.DMA,
        ],
    )
    def gk(emb_hbm, idx_hbm, out_hbm, idx_v, rows_v, out_v, sem):
        iota16 = lax.iota(jnp.int32, 16)
        b = lax.axis_index("s") * nc + lax.axis_index("c")
        pltpu.sync_copy(idx_hbm.at[pl.ds(b * HW, HW)], idx_v)
        pltpu.async_copy(emb_hbm.at[idx_v], rows_v, sem).wait()

        def k0_body(k0, carry):
            for g in range(4):
                rows_g = k0 * 64 + 16 * g + iota16
                for m0 in range(64):
                    cols = jnp.full((16,), m0, jnp.int32)
                    v = plsc.load_gather(rows_v, [rows_g, cols])
                    out_v[m0, pl.ds(16 * g, 16)] = v
            pltpu.sync_copy(out_v, out_hbm.at[b, pl.ds(k0 * 64, 64)])
            return carry

        lax.fori_loop(0, 9, k0_body, 0)

    return gk(emb_pad, idxp)


def kernel(z, emb):
    z_flat = jnp.transpose(z, (0, 2, 3, 1)).reshape(ROWS, DIM)
    embT = emb.T
    idx3, dsum3 = _tc_argmin(z_flat, embT)
    idxp = (idx3.reshape(32, 64, 9).transpose(0, 2, 1)).reshape(ROWS)
    emb_pad = jnp.pad(emb, ((0, 0), (0, 128 - DIM)))
    out_sc = _sc_gather_view(emb_pad, idxp)
    quantized_st = (out_sc[:, :, :DIM]
                    .reshape(32, 24, 24, DIM)
                    .transpose(0, 3, 1, 2))
    total = jnp.sum(dsum3[:, 0, 0])
    codebook_loss = (total / jnp.float32(ROWS * DIM)).astype(jnp.float32)
    commitment_loss = (jnp.float32(BETA_C) * codebook_loss).astype(jnp.float32)
    return (quantized_st, commitment_loss, codebook_loss,
            jnp.float32(0.0), jnp.float32(0.0))


# speedup vs baseline: 2.1748x; 1.1286x over previous
import functools

import jax
import jax.numpy as jnp
from jax import lax
from jax.experimental import pallas as pl
from jax.experimental.pallas import tpu as pltpu
from jax.experimental.pallas import tpu_sc as plsc

N_CODES = 1024
DIM = 64
ROWS = 18432
HW = 576
BLK = 2304
NB = ROWS // BLK
BETA_C = 0.25


def _argmin_body(zf_ref, embTs_ref, idx_ref, dsum_ref):
    zb = zf_ref[...]
    embTs = embTs_ref[...]
    m = lax.dot_general(zb, embTs,
                        dimension_numbers=(((1,), (0,)), ((), ())),
                        preferred_element_type=jnp.float32)
    z_sq = jnp.sum(zb * zb, axis=1, keepdims=True)
    e_sq = jnp.sum(embTs * embTs, axis=0, keepdims=True) * 0.25
    dist = z_sq + m + e_sq
    dmin = jnp.min(dist, axis=1, keepdims=True)
    iota = lax.broadcasted_iota(jnp.int32, dist.shape, 1).astype(jnp.float32)
    cand = jnp.where(dist == dmin, iota, jnp.float32(N_CODES))
    idx_ref[0, 0, :] = jnp.min(cand, axis=1).astype(jnp.int32)
    dsum_ref[0, 0, :] = jnp.broadcast_to(jnp.sum(dmin), (128,))


def _tc_argmin(z_flat, embTs):
    return pl.pallas_call(
        _argmin_body,
        grid=(NB,),
        in_specs=[
            pl.BlockSpec((BLK, DIM), lambda i: (i, 0)),
            pl.BlockSpec((DIM, N_CODES), lambda i: (0, 0)),
        ],
        out_specs=[
            pl.BlockSpec((1, 1, BLK), lambda i: (i, 0, 0)),
            pl.BlockSpec((1, 1, 128), lambda i: (i, 0, 0)),
        ],
        out_shape=[
            jax.ShapeDtypeStruct((NB, 1, BLK), jnp.int32),
            jax.ShapeDtypeStruct((NB, 1, 128), jnp.float32),
        ],
    )(z_flat, embTs)


def _sc_gather_view(emb_pad, idxp):
    info = plsc.get_sparse_core_info()
    nc, ns = info.num_cores, info.num_subcores
    assert nc * ns == 32
    mesh = plsc.VectorSubcoreMesh(core_axis_name="c", subcore_axis_name="s")

    @functools.partial(
        pl.kernel, mesh=mesh,
        compiler_params=pltpu.CompilerParams(needs_layout_passes=False),
        out_type=jax.ShapeDtypeStruct((32, HW, 128), jnp.float32),
        scratch_types=[
            pltpu.VMEM((HW,), jnp.int32),
            pltpu.VMEM((HW, 128), jnp.float32),
            pltpu.VMEM((64, 128), jnp.float32),
            pltpu.SemaphoreType.DMA,
        ],
    )
    def gk(emb_hbm, idx_hbm, out_hbm, idx_v, rows_v, out_v, sem):
        iota16 = lax.iota(jnp.int32, 16)
        b = lax.axis_index("s") * nc + lax.axis_index("c")
        pltpu.sync_copy(idx_hbm.at[pl.ds(b * HW, HW)], idx_v)
        pltpu.async_copy(emb_hbm.at[idx_v], rows_v, sem).wait()

        def k0_body(k0, carry):
            rows = [k0 * 64 + 16 * g + iota16 for g in range(4)]

            @plsc.parallel_loop(0, 64, unroll=8)
            def m0_body(m0):
                cols = jnp.full((16,), m0, jnp.int32)
                for g in range(4):
                    v = plsc.load_gather(rows_v, [rows[g], cols])
                    out_v[m0, pl.ds(16 * g, 16)] = v

            pltpu.sync_copy(out_v, out_hbm.at[b, pl.ds(k0 * 64, 64)])
            return carry

        lax.fori_loop(0, 9, k0_body, 0)

    return gk(emb_pad, idxp)


def kernel(z, emb):
    z_flat = jnp.transpose(z, (0, 2, 3, 1)).reshape(ROWS, DIM)
    embTs = emb.T * jnp.float32(-2.0)
    idx3, dsum3 = _tc_argmin(z_flat, embTs)
    idxp = (idx3.reshape(32, 64, 9).transpose(0, 2, 1)).reshape(ROWS)
    emb_pad = jnp.pad(emb, ((0, 0), (0, 128 - DIM)))
    out_sc = _sc_gather_view(emb_pad, idxp)
    quantized_st = (out_sc[:, :, :DIM]
                    .reshape(32, 24, 24, DIM)
                    .transpose(0, 3, 1, 2))
    total = jnp.sum(dsum3[:, 0, 0])
    codebook_loss = (total / jnp.float32(ROWS * DIM)).astype(jnp.float32)
    commitment_loss = (jnp.float32(BETA_C) * codebook_loss).astype(jnp.float32)
    return (quantized_st, commitment_loss, codebook_loss,
            jnp.float32(0.0), jnp.float32(0.0))


# speedup vs baseline: 2.2278x; 1.0244x over previous
import functools

import jax
import jax.numpy as jnp
from jax import lax
from jax.experimental import pallas as pl
from jax.experimental.pallas import tpu as pltpu
from jax.experimental.pallas import tpu_sc as plsc

N_CODES = 1024
DIM = 64
ROWS = 18432
HW = 576
BLK = 2304
NB = ROWS // BLK
BETA_C = 0.25


def _argmin_body(zf_ref, embTs_ref, esq_ref, iota_ref, idx_ref, dsum_ref):
    zb = zf_ref[...]
    m = lax.dot_general(zb, embTs_ref[...],
                        dimension_numbers=(((1,), (0,)), ((), ())),
                        preferred_element_type=jnp.float32)
    z_sq = jnp.sum(zb * zb, axis=1, keepdims=True)
    dist = z_sq + m + esq_ref[...]
    dmin = jnp.min(dist, axis=1, keepdims=True)
    cand = jnp.where(dist == dmin, iota_ref[...], jnp.float32(N_CODES))
    idx_ref[0, 0, :] = jnp.min(cand, axis=1).astype(jnp.int32)
    dsum_ref[0, 0, :] = jnp.broadcast_to(jnp.sum(dmin), (128,))


def _tc_argmin(z_flat, embTs, e_sq_row, iota_row):
    return pl.pallas_call(
        _argmin_body,
        grid=(NB,),
        in_specs=[
            pl.BlockSpec((BLK, DIM), lambda i: (i, 0)),
            pl.BlockSpec((DIM, N_CODES), lambda i: (0, 0)),
            pl.BlockSpec((1, N_CODES), lambda i: (0, 0)),
            pl.BlockSpec((1, N_CODES), lambda i: (0, 0)),
        ],
        out_specs=[
            pl.BlockSpec((1, 1, BLK), lambda i: (i, 0, 0)),
            pl.BlockSpec((1, 1, 128), lambda i: (i, 0, 0)),
        ],
        out_shape=[
            jax.ShapeDtypeStruct((NB, 1, BLK), jnp.int32),
            jax.ShapeDtypeStruct((NB, 1, 128), jnp.float32),
        ],
    )(z_flat, embTs, e_sq_row, iota_row)


def _sc_gather_view(emb_pad, idxp):
    info = plsc.get_sparse_core_info()
    nc, ns = info.num_cores, info.num_subcores
    assert nc * ns == 32
    mesh = plsc.VectorSubcoreMesh(core_axis_name="c", subcore_axis_name="s")

    @functools.partial(
        pl.kernel, mesh=mesh,
        compiler_params=pltpu.CompilerParams(needs_layout_passes=False),
        out_type=jax.ShapeDtypeStruct((32, HW, 128), jnp.float32),
        scratch_types=[
            pltpu.VMEM((HW,), jnp.int32),
            pltpu.VMEM((HW, 128), jnp.float32),
            pltpu.VMEM((2, 64, 128), jnp.float32),
            pltpu.SemaphoreType.DMA,
            pltpu.SemaphoreType.DMA,
        ],
    )
    def gk(emb_hbm, idx_hbm, out_hbm, idx_v, rows_v, out_v, sem, osem):
        iota16 = lax.iota(jnp.int32, 16)
        b = lax.axis_index("s") * nc + lax.axis_index("c")
        pltpu.sync_copy(idx_hbm.at[pl.ds(b * HW, HW)], idx_v)
        pltpu.async_copy(emb_hbm.at[idx_v], rows_v, sem).wait()

        def k0_body(k0, carry):
            slot = lax.rem(k0, 2)
            rows = [k0 * 64 + 16 * g + iota16 for g in range(4)]

            @plsc.parallel_loop(0, 64, unroll=8)
            def m0_body(m0):
                cols = jnp.full((16,), m0, jnp.int32)
                for g in range(4):
                    v = plsc.load_gather(rows_v, [rows[g], cols])
                    out_v[slot, m0, pl.ds(16 * g, 16)] = v

            pltpu.async_copy(out_v.at[slot],
                             out_hbm.at[b, pl.ds(k0 * 64, 64)], osem)

            @pl.when(k0 >= 1)
            def _():
                pltpu.make_async_copy(
                    out_v.at[0], out_hbm.at[b, pl.ds(0, 64)], osem).wait()

            return carry

        lax.fori_loop(0, 9, k0_body, 0)
        pltpu.make_async_copy(
            out_v.at[0], out_hbm.at[b, pl.ds(0, 64)], osem).wait()

    return gk(emb_pad, idxp)


def kernel(z, emb):
    z_flat = jnp.transpose(z, (0, 2, 3, 1)).reshape(ROWS, DIM)
    embTs = emb.T * jnp.float32(-2.0)
    e_sq_row = jnp.sum(emb * emb, axis=1).reshape(1, N_CODES)
    iota_row = jnp.arange(N_CODES, dtype=jnp.float32).reshape(1, N_CODES)
    idx3, dsum3 = _tc_argmin(z_flat, embTs, e_sq_row, iota_row)
    idxp = (idx3.reshape(32, 64, 9).transpose(0, 2, 1)).reshape(ROWS)
    emb_pad = jnp.pad(emb, ((0, 0), (0, 128 - DIM)))
    out_sc = _sc_gather_view(emb_pad, idxp)
    quantized_st = (out_sc[:, :, :DIM]
                    .reshape(32, 24, 24, DIM)
                    .transpose(0, 3, 1, 2))
    total = jnp.sum(dsum3[:, 0, 0])
    codebook_loss = (total / jnp.float32(ROWS * DIM)).astype(jnp.float32)
    commitment_loss = (jnp.float32(BETA_C) * codebook_loss).astype(jnp.float32)
    return (quantized_st, commitment_loss, codebook_loss,
            jnp.float32(0.0), jnp.float32(0.0))


# speedup vs baseline: 2.2375x; 1.0043x over previous
import functools

import jax
import jax.numpy as jnp
from jax import lax
from jax.experimental import pallas as pl
from jax.experimental.pallas import tpu as pltpu
from jax.experimental.pallas import tpu_sc as plsc

N_CODES = 1024
DIM = 64
ROWS = 18432
HW = 576
BLK = 3072
NB = ROWS // BLK
BETA_C = 0.25


def _argmin_body(zf_ref, embTs_ref, esq_ref, iota_ref, idx_ref, dsum_ref):
    zb = zf_ref[...]
    m = lax.dot_general(zb, embTs_ref[...],
                        dimension_numbers=(((1,), (0,)), ((), ())),
                        preferred_element_type=jnp.float32)
    z_sq = jnp.sum(zb * zb, axis=1, keepdims=True)
    dist = z_sq + m + esq_ref[...]
    dmin = jnp.min(dist, axis=1, keepdims=True)
    cand = jnp.where(dist == dmin, iota_ref[...], jnp.float32(N_CODES))
    idx_ref[...] = jnp.min(cand, axis=1).astype(jnp.int32)
    dsum_ref[0, 0, :] = jnp.broadcast_to(jnp.sum(dmin), (128,))


def _tc_argmin(z_flat, embTs, e_sq_row, iota_row):
    return pl.pallas_call(
        _argmin_body,
        grid=(NB,),
        in_specs=[
            pl.BlockSpec((BLK, DIM), lambda i: (i, 0)),
            pl.BlockSpec((DIM, N_CODES), lambda i: (0, 0)),
            pl.BlockSpec((1, N_CODES), lambda i: (0, 0)),
            pl.BlockSpec((1, N_CODES), lambda i: (0, 0)),
        ],
        out_specs=[
            pl.BlockSpec((BLK,), lambda i: (i,)),
            pl.BlockSpec((1, 1, 128), lambda i: (i, 0, 0)),
        ],
        out_shape=[
            jax.ShapeDtypeStruct((ROWS,), jnp.int32),
            jax.ShapeDtypeStruct((NB, 1, 128), jnp.float32),
        ],
    )(z_flat, embTs, e_sq_row, iota_row)


def _sc_gather_view(emb_pad, idxp):
    info = plsc.get_sparse_core_info()
    nc, ns = info.num_cores, info.num_subcores
    assert nc * ns == 32
    mesh = plsc.VectorSubcoreMesh(core_axis_name="c", subcore_axis_name="s")

    @functools.partial(
        pl.kernel, mesh=mesh,
        compiler_params=pltpu.CompilerParams(needs_layout_passes=False),
        out_type=jax.ShapeDtypeStruct((32, HW, 128), jnp.float32),
        scratch_types=[
            pltpu.VMEM((HW,), jnp.int32),
            pltpu.VMEM((HW, 128), jnp.float32),
            pltpu.VMEM((2, 64, 128), jnp.float32),
            pltpu.SemaphoreType.DMA,
            pltpu.SemaphoreType.DMA,
        ],
    )
    def gk(emb_hbm, idx_hbm, out_hbm, idx_v, rows_v, out_v, sem, osem):
        iota16 = lax.iota(jnp.int32, 16)
        b = lax.axis_index("s") * nc + lax.axis_index("c")
        pltpu.sync_copy(idx_hbm.at[pl.ds(b * HW, HW)], idx_v)
        pltpu.async_copy(emb_hbm.at[idx_v], rows_v, sem).wait()

        def k0_body(k0, carry):
            slot = lax.rem(k0, 2)
            rows = [k0 * 64 + 16 * g + iota16 for g in range(4)]

            @plsc.parallel_loop(0, 64, unroll=8)
            def m0_body(m0):
                cols = jnp.full((16,), m0, jnp.int32)
                for g in range(4):
                    v = plsc.load_gather(rows_v, [rows[g], cols])
                    out_v[slot, m0, pl.ds(16 * g, 16)] = v

            pltpu.async_copy(out_v.at[slot],
                             out_hbm.at[b, pl.ds(k0 * 64, 64)], osem)

            @pl.when(k0 >= 1)
            def _():
                pltpu.make_async_copy(
                    out_v.at[0], out_hbm.at[b, pl.ds(0, 64)], osem).wait()

            return carry

        lax.fori_loop(0, 9, k0_body, 0)
        pltpu.make_async_copy(
            out_v.at[0], out_hbm.at[b, pl.ds(0, 64)], osem).wait()

    return gk(emb_pad, idxp)


def kernel(z, emb):
    z_flat = jnp.transpose(z, (0, 2, 3, 1)).reshape(ROWS, DIM)
    embTs = emb.T * jnp.float32(-2.0)
    e_sq_row = jnp.sum(emb * emb, axis=1).reshape(1, N_CODES)
    iota_row = jnp.arange(N_CODES, dtype=jnp.float32).reshape(1, N_CODES)
    idx1, dsum3 = _tc_argmin(z_flat, embTs, e_sq_row, iota_row)
    idxp = (idx1.reshape(32, 64, 9).transpose(0, 2, 1)).reshape(ROWS)
    emb_pad = jnp.pad(emb, ((0, 0), (0, 128 - DIM)))
    out_sc = _sc_gather_view(emb_pad, idxp)
    quantized_st = (out_sc[:, :, :DIM]
                    .reshape(32, 24, 24, DIM)
                    .transpose(0, 3, 1, 2))
    total = jnp.sum(dsum3[:, 0, 0])
    codebook_loss = (total / jnp.float32(ROWS * DIM)).astype(jnp.float32)
    commitment_loss = (jnp.float32(BETA_C) * codebook_loss).astype(jnp.float32)
    return (quantized_st, commitment_loss, codebook_loss,
            jnp.float32(0.0), jnp.float32(0.0))


# speedup vs baseline: 2.2392x; 1.0008x over previous
import functools

import jax
import jax.numpy as jnp
from jax import lax
from jax.experimental import pallas as pl
from jax.experimental.pallas import tpu as pltpu
from jax.experimental.pallas import tpu_sc as plsc

N_CODES = 1024
DIM = 64
ROWS = 18432
HW = 576
BLK = 3072
NB = ROWS // BLK
BETA_C = 0.25


def _argmin_body(zf_ref, embTs_ref, esq_ref, iota_ref, idx_ref, dsum_ref):
    zb = zf_ref[...]
    m = lax.dot_general(zb, embTs_ref[...],
                        dimension_numbers=(((1,), (0,)), ((), ())),
                        preferred_element_type=jnp.float32)
    z_sq = jnp.sum(zb * zb, axis=1, keepdims=True)
    dist = z_sq + m + esq_ref[...]
    dmin = jnp.min(dist, axis=1, keepdims=True)
    cand = jnp.where(dist == dmin, iota_ref[...], jnp.float32(N_CODES))
    idx_ref[...] = jnp.min(cand, axis=1).astype(jnp.int32)
    dsum_ref[0, 0, :] = jnp.broadcast_to(jnp.sum(dmin), (128,))


def _tc_argmin(z_flat, embTs, e_sq_row, iota_row):
    return pl.pallas_call(
        _argmin_body,
        grid=(NB,),
        in_specs=[
            pl.BlockSpec((BLK, DIM), lambda i: (i, 0)),
            pl.BlockSpec((DIM, N_CODES), lambda i: (0, 0)),
            pl.BlockSpec((1, N_CODES), lambda i: (0, 0)),
            pl.BlockSpec((1, N_CODES), lambda i: (0, 0)),
        ],
        out_specs=[
            pl.BlockSpec((BLK,), lambda i: (i,)),
            pl.BlockSpec((1, 1, 128), lambda i: (i, 0, 0)),
        ],
        out_shape=[
            jax.ShapeDtypeStruct((ROWS,), jnp.int32),
            jax.ShapeDtypeStruct((NB, 1, 128), jnp.float32),
        ],
    )(z_flat, embTs, e_sq_row, iota_row)


def _sc_gather_view(emb_pad, idxp):
    info = plsc.get_sparse_core_info()
    nc, ns = info.num_cores, info.num_subcores
    assert nc * ns == 32
    mesh = plsc.VectorSubcoreMesh(core_axis_name="c", subcore_axis_name="s")

    @functools.partial(
        pl.kernel, mesh=mesh,
        compiler_params=pltpu.CompilerParams(needs_layout_passes=False),
        out_type=jax.ShapeDtypeStruct((32, HW, 128), jnp.float32),
        scratch_types=[
            pltpu.VMEM((HW,), jnp.int32),
            pltpu.VMEM((HW, 128), jnp.float32),
            pltpu.VMEM((2, 64, 128), jnp.float32),
            pltpu.SemaphoreType.DMA,
            pltpu.SemaphoreType.DMA,
        ],
    )
    def gk(emb_hbm, idx_hbm, out_hbm, idx_v, rows_v, out_v, sem, osem):
        iota16 = lax.iota(jnp.int32, 16)
        b = lax.axis_index("s") * nc + lax.axis_index("c")
        pltpu.sync_copy(idx_hbm.at[pl.ds(b * HW, HW)], idx_v)
        pltpu.async_copy(emb_hbm.at[idx_v], rows_v, sem).wait()

        def k0_body(k0, carry):
            slot = lax.rem(k0, 2)
            rows = [k0 * 64 + 16 * g + iota16 for g in range(4)]

            @plsc.parallel_loop(0, 64, unroll=4)
            def m0_body(m0):
                cols = jnp.full((16,), m0, jnp.int32)
                for g in range(4):
                    v = plsc.load_gather(rows_v, [rows[g], cols])
                    out_v[slot, m0, pl.ds(16 * g, 16)] = v

            pltpu.async_copy(out_v.at[slot],
                             out_hbm.at[b, pl.ds(k0 * 64, 64)], osem)

            @pl.when(k0 >= 1)
            def _():
                pltpu.make_async_copy(
                    out_v.at[0], out_hbm.at[b, pl.ds(0, 64)], osem).wait()

            return carry

        lax.fori_loop(0, 9, k0_body, 0)
        pltpu.make_async_copy(
            out_v.at[0], out_hbm.at[b, pl.ds(0, 64)], osem).wait()

    return gk(emb_pad, idxp)


def kernel(z, emb):
    z_flat = jnp.transpose(z, (0, 2, 3, 1)).reshape(ROWS, DIM)
    embTs = emb.T * jnp.float32(-2.0)
    e_sq_row = jnp.sum(emb * emb, axis=1).reshape(1, N_CODES)
    iota_row = jnp.arange(N_CODES, dtype=jnp.float32).reshape(1, N_CODES)
    idx1, dsum3 = _tc_argmin(z_flat, embTs, e_sq_row, iota_row)
    idxp = (idx1.reshape(32, 64, 9).transpose(0, 2, 1)).reshape(ROWS)
    emb_pad = jnp.pad(emb, ((0, 0), (0, 128 - DIM)))
    out_sc = _sc_gather_view(emb_pad, idxp)
    quantized_st = (out_sc[:, :, :DIM]
                    .reshape(32, 24, 24, DIM)
                    .transpose(0, 3, 1, 2))
    total = jnp.sum(dsum3[:, 0, 0])
    codebook_loss = (total / jnp.float32(ROWS * DIM)).astype(jnp.float32)
    commitment_loss = (jnp.float32(BETA_C) * codebook_loss).astype(jnp.float32)
    return (quantized_st, commitment_loss, codebook_loss,
            jnp.float32(0.0), jnp.float32(0.0))
